# Initial kernel scaffold; baseline (speedup 1.0000x reference)
import functools

import jax
import jax.numpy as jnp
from jax import lax
from jax.experimental import pallas as pl
from jax.experimental.pallas import tpu as pltpu
from jax.experimental.pallas import tpu_sc as plsc

EMBED = 32
N_PIECE = 13
N_POS = 64
BATCH = 16384
TOTAL_ROWS = BATCH * N_POS
NUM_WORKERS = 32
ROWS_PER_W = TOTAL_ROWS // NUM_WORKERS
CHUNK = 128
N_CHUNK = ROWS_PER_W // CHUNK
LANES = 16


def _build_combined(piece_table, position_table):
    def body(piece_ref, pos_ref, out_ref):
        out_ref[...] = piece_ref[...][:, None, :] + pos_ref[...][None, :, :]

    out = pl.pallas_call(
        body,
        out_shape=jax.ShapeDtypeStruct((N_PIECE, N_POS, EMBED), jnp.float32),
    )(piece_table, position_table)
    return out.reshape(N_PIECE * N_POS, EMBED)


def _sc_lookup(board_flat, combined):
    mesh = plsc.VectorSubcoreMesh(core_axis_name="c", subcore_axis_name="s")

    @functools.partial(
        pl.kernel,
        out_type=jax.ShapeDtypeStruct((TOTAL_ROWS, EMBED), jnp.float32),
        mesh=mesh,
        scratch_types=[
            pltpu.VMEM((ROWS_PER_W,), jnp.int32),
            pltpu.VMEM((CHUNK,), jnp.int32),
            pltpu.VMEM((CHUNK,), jnp.int32),
            pltpu.VMEM((CHUNK, EMBED), jnp.float32),
            pltpu.VMEM((CHUNK, EMBED), jnp.float32),
            pltpu.SemaphoreType.DMA,
            pltpu.SemaphoreType.DMA,
            pltpu.SemaphoreType.DMA,
            pltpu.SemaphoreType.DMA,
        ],
    )
    def k(board_hbm, comb_hbm, out_hbm, bbuf, ib0, ib1, rb0, rb1,
          gs0, gs1, os0, os1):
        wid = lax.axis_index("s") * 2 + lax.axis_index("c")
        base = wid * ROWS_PER_W

        pltpu.sync_copy(board_hbm.at[pl.ds(base, ROWS_PER_W)], bbuf)

        lane = lax.broadcasted_iota(jnp.int32, (LANES,), 0)
        pos_vecs = [lane + (v * LANES) % N_POS for v in range(CHUNK // LANES)]

        def compute_idx(g, ibuf):
            off = g * CHUNK
            for v in range(CHUNK // LANES):
                bv = bbuf[pl.ds(off + v * LANES, LANES)]
                ibuf[pl.ds(v * LANES, LANES)] = bv * N_POS + pos_vecs[v]

        def out_slice(g):
            return out_hbm.at[pl.ds(base + g * CHUNK, CHUNK)]

        def loop_body(i, carry):
            g0 = i * 2
            g1 = g0 + 1

            @pl.when(i > 0)
            def _wait_prev_scatters():
                pltpu.make_async_copy(rb0, out_slice(g0), os0).wait()
                pltpu.make_async_copy(rb1, out_slice(g1), os1).wait()

            compute_idx(g0, ib0)
            pltpu.async_copy(comb_hbm.at[ib0], rb0, gs0)
            compute_idx(g1, ib1)
            pltpu.async_copy(comb_hbm.at[ib1], rb1, gs1)

            pltpu.make_async_copy(comb_hbm.at[ib0], rb0, gs0).wait()
            pltpu.async_copy(rb0, out_slice(g0), os0)
            pltpu.make_async_copy(comb_hbm.at[ib1], rb1, gs1).wait()
            pltpu.async_copy(rb1, out_slice(g1), os1)
            return carry

        lax.fori_loop(0, N_CHUNK // 2, loop_body, 0)
        pltpu.make_async_copy(rb0, out_slice(0), os0).wait()
        pltpu.make_async_copy(rb1, out_slice(0), os1).wait()

    return k(board_flat, combined)


def kernel(board, piece_table, position_table):
    board_flat = board.reshape(TOTAL_ROWS).astype(jnp.int32)
    combined = _build_combined(piece_table, position_table)
    out = _sc_lookup(board_flat, combined)
    return out.reshape(BATCH, 8, 8, EMBED)


# speedup vs baseline: 3.7017x; 3.7017x over previous
import functools

import jax
import jax.numpy as jnp
from jax import lax
from jax.experimental import pallas as pl
from jax.experimental.pallas import tpu as pltpu
from jax.experimental.pallas import tpu_sc as plsc

EMBED = 32
N_PIECE = 13
N_POS = 64
BATCH = 16384
TOTAL_ROWS = BATCH * N_POS
NUM_WORKERS = 32
ROWS_PER_W = TOTAL_ROWS // NUM_WORKERS
CHUNK = 128
N_CHUNK = ROWS_PER_W // CHUNK
LANES = 16


def _build_combined(piece_table, position_table):
    def body(piece_ref, pos_ref, out_ref):
        out_ref[...] = piece_ref[...][:, None, :] + pos_ref[...][None, :, :]

    out = pl.pallas_call(
        body,
        out_shape=jax.ShapeDtypeStruct((N_PIECE, N_POS, EMBED), jnp.float32),
    )(piece_table, position_table)
    return out.reshape(N_PIECE * N_POS, EMBED)


def _sc_lookup(board_flat, combined):
    mesh = plsc.VectorSubcoreMesh(core_axis_name="c", subcore_axis_name="s")

    @functools.partial(
        pl.kernel,
        out_type=jax.ShapeDtypeStruct((TOTAL_ROWS, EMBED), jnp.float32),
        mesh=mesh,
        compiler_params=pltpu.CompilerParams(use_tc_tiling_on_sc=False),
        scratch_types=[
            pltpu.VMEM((ROWS_PER_W,), jnp.int32),
            pltpu.VMEM((CHUNK,), jnp.int32),
            pltpu.VMEM((CHUNK,), jnp.int32),
            pltpu.VMEM((CHUNK, EMBED), jnp.float32),
            pltpu.VMEM((CHUNK, EMBED), jnp.float32),
            pltpu.SemaphoreType.DMA,
            pltpu.SemaphoreType.DMA,
            pltpu.SemaphoreType.DMA,
            pltpu.SemaphoreType.DMA,
        ],
    )
    def k(board_hbm, comb_hbm, out_hbm, bbuf, ib0, ib1, rb0, rb1,
          gs0, gs1, os0, os1):
        wid = lax.axis_index("s") * 2 + lax.axis_index("c")
        base = wid * ROWS_PER_W

        pltpu.sync_copy(board_hbm.at[pl.ds(base, ROWS_PER_W)], bbuf)

        lane = lax.broadcasted_iota(jnp.int32, (LANES,), 0)
        pos_vecs = [lane + (v * LANES) % N_POS for v in range(CHUNK // LANES)]

        def compute_idx(g, ibuf):
            off = g * CHUNK
            for v in range(CHUNK // LANES):
                bv = bbuf[pl.ds(off + v * LANES, LANES)]
                ibuf[pl.ds(v * LANES, LANES)] = bv * N_POS + pos_vecs[v]

        def out_slice(g):
            return out_hbm.at[pl.ds(base + g * CHUNK, CHUNK)]

        def loop_body(i, carry):
            g0 = i * 2
            g1 = g0 + 1

            @pl.when(i > 0)
            def _wait_prev_scatters():
                pltpu.make_async_copy(rb0, out_slice(g0), os0).wait()
                pltpu.make_async_copy(rb1, out_slice(g1), os1).wait()

            compute_idx(g0, ib0)
            pltpu.async_copy(comb_hbm.at[ib0], rb0, gs0)
            compute_idx(g1, ib1)
            pltpu.async_copy(comb_hbm.at[ib1], rb1, gs1)

            pltpu.make_async_copy(comb_hbm.at[ib0], rb0, gs0).wait()
            pltpu.async_copy(rb0, out_slice(g0), os0)
            pltpu.make_async_copy(comb_hbm.at[ib1], rb1, gs1).wait()
            pltpu.async_copy(rb1, out_slice(g1), os1)
            return carry

        lax.fori_loop(0, N_CHUNK // 2, loop_body, 0)
        pltpu.make_async_copy(rb0, out_slice(0), os0).wait()
        pltpu.make_async_copy(rb1, out_slice(0), os1).wait()

    return k(board_flat, combined)


def kernel(board, piece_table, position_table):
    board_flat = board.reshape(TOTAL_ROWS).astype(jnp.int32)
    combined = _build_combined(piece_table, position_table)
    out = _sc_lookup(board_flat, combined)
    return out.reshape(BATCH, 8, 8, EMBED)


# speedup vs baseline: 53.6601x; 14.4961x over previous
import functools

import jax
import jax.numpy as jnp
from jax import lax
from jax.experimental import pallas as pl
from jax.experimental.pallas import tpu as pltpu
from jax.experimental.pallas import tpu_sc as plsc

EMBED = 32
N_PIECE = 13
N_PIECE_PAD = 16
N_POS = 64
BATCH = 16384
BCHUNK = 1024
BQUARTER = BATCH // 4
NCH = BQUARTER // BCHUNK
LANES = 16
TROW = EMBED * N_PIECE_PAD


def _build_combined(piece_pad_t, position_table):
    def body(piece_ref, pos_ref, out_ref):
        out_ref[...] = (
            lax.broadcast_in_dim(piece_ref[...], (N_POS, EMBED, N_PIECE_PAD), (1, 2))
            + lax.broadcast_in_dim(pos_ref[...], (N_POS, EMBED, N_PIECE_PAD), (0, 1))
        )

    return pl.pallas_call(
        body,
        out_shape=jax.ShapeDtypeStruct((N_POS, EMBED, N_PIECE_PAD), jnp.float32),
    )(piece_pad_t, position_table)


def _sc_lookup(board_t, comb_flat):
    mesh = plsc.VectorSubcoreMesh(core_axis_name="c", subcore_axis_name="s")

    @functools.partial(
        pl.kernel,
        out_type=jax.ShapeDtypeStruct((8, 8, EMBED, BATCH), jnp.float32),
        mesh=mesh,
        compiler_params=pltpu.CompilerParams(use_tc_tiling_on_sc=True),
        scratch_types=[
            pltpu.VMEM((8 * TROW,), jnp.float32),
            pltpu.VMEM((8, BCHUNK), jnp.int32),
            pltpu.VMEM((EMBED, BCHUNK), jnp.float32),
            pltpu.VMEM((EMBED, BCHUNK), jnp.float32),
            pltpu.SemaphoreType.DMA,
            pltpu.SemaphoreType.DMA,
        ],
    )
    def k(board_hbm, comb_hbm, out_hbm, tbuf, bb0, ob0, ob1, os0, os1):
        wid = lax.axis_index("s") * 2 + lax.axis_index("c")
        r = wid // 4
        bq = wid % 4
        bbase = bq * BQUARTER
        pltpu.sync_copy(comb_hbm.at[pl.ds(r * 8 * TROW, 8 * TROW)], tbuf)

        def fill_chunk(bb, ob, c):
            tvecs = [
                tbuf[pl.ds((c * EMBED + d) * N_PIECE_PAD, LANES)] for d in range(EMBED)
            ]

            dnums = lax.GatherDimensionNumbers(
                offset_dims=(), collapsed_slice_dims=(0,), start_index_map=(0,)
            )

            def body(kk, carry):
                bv = bb[c, pl.ds(kk * LANES, LANES)]
                idx = bv.reshape(LANES, 1)
                for d in range(EMBED):
                    ob[d, pl.ds(kk * LANES, LANES)] = lax.gather(
                        tvecs[d],
                        idx,
                        dimension_numbers=dnums,
                        slice_sizes=(1,),
                        mode=lax.GatherScatterMode.PROMISE_IN_BOUNDS,
                    )
                return carry

            lax.fori_loop(0, BCHUNK // LANES, body, 0)

        def out_slice(c, g):
            return out_hbm.at[r, c, :, pl.ds(bbase + g * BCHUNK, BCHUNK)]

        obs = (ob0, ob1)
        sems = (os0, os1)

        def loop_body(g, carry):
            bb = bb0
            pltpu.sync_copy(
                board_hbm.at[r, :, pl.ds(bbase + g * BCHUNK, BCHUNK)], bb
            )
            for c in range(8):
                buf = c % 2

                def _wait():
                    pltpu.make_async_copy(obs[buf], out_slice(0, 0), sems[buf]).wait()

                if c < 2:
                    pl.when(g > 0)(_wait)
                else:
                    _wait()
                fill_chunk(bb, obs[buf], c)
                pltpu.async_copy(obs[buf], out_slice(c, g), sems[buf])
            return carry

        lax.fori_loop(0, NCH, loop_body, 0)
        pltpu.make_async_copy(ob0, out_slice(0, 0), os0).wait()
        pltpu.make_async_copy(ob1, out_slice(0, 0), os1).wait()

    return k(board_t, comb_flat)


def kernel(board, piece_table, position_table):
    board_t = jnp.transpose(board.astype(jnp.int32), (1, 2, 0))
    piece_pad_t = jnp.pad(piece_table, ((0, N_PIECE_PAD - N_PIECE), (0, 0))).T
    comb = _build_combined(piece_pad_t, position_table)
    comb_flat = comb.reshape(N_POS * EMBED * N_PIECE_PAD)
    out_t = _sc_lookup(board_t, comb_flat)
    return jnp.transpose(out_t, (3, 0, 1, 2))
